# baseline (speedup 1.0000x reference)
import functools

import jax
import jax.numpy as jnp
from jax import lax
from jax.experimental import pallas as pl
from jax.experimental.pallas import tpu as pltpu
from jax.experimental.pallas import tpu_sc as plsc

_L = 16
_NC = 2
_NS = 16
_NW = _NC * _NS

_EF = 1.5
_TS = 256.0
_ML = 3.0

_GDN = lax.GatherDimensionNumbers(
    offset_dims=(), collapsed_slice_dims=(0,), start_index_map=(0,))


def _vgather(v, idx):
  return lax.gather(v, idx[:, None], _GDN, (1,),
                    mode=lax.GatherScatterMode.PROMISE_IN_BOUNDS)


def _floorv(v):
  t = v.astype(jnp.int32).astype(jnp.float32)
  return jnp.where(t > v, t - 1.0, t)


def _ceilv(v):
  t = v.astype(jnp.int32).astype(jnp.float32)
  return jnp.where(t < v, t + 1.0, t)


def _align16(x, y, bw, bh, H, W):
  w = _ceilv(bw * _EF)
  h = _ceilv(bh * _EF)
  ix = _floorv(x - w * 0.5)
  cx = ix < 0.0
  w = jnp.where(cx, w + ix, w)
  ix = jnp.where(cx, 0.0, ix)
  iy = _floorv(y - h * 0.5)
  cy = iy < 0.0
  h = jnp.where(cy, h + iy, h)
  iy = jnp.where(cy, 0.0, iy)
  w = jnp.maximum(w, _ML)
  h = jnp.maximum(h, _ML)
  iw = W - ix
  iw = jnp.where(w < iw, w, iw)
  ih = H - iy
  ih = jnp.where(h < ih, h, ih)
  sx = iw < _ML
  iw = jnp.where(sx, _ML, iw)
  ix = jnp.where(sx, W - _ML, ix)
  sy = ih < _ML
  ih = jnp.where(sy, _ML, ih)
  iy = jnp.where(sy, H - _ML, iy)
  nidx = iw > ih
  th = jnp.where(nidx, _TS, _TS * ih / iw)
  tw = jnp.where(nidx, _TS * iw / ih, _TS)
  return ix, iy, iw, ih, tw, th


@functools.lru_cache(maxsize=None)
def _make_sc_kernel(n, bpw, H, W):
  ngroups = bpw // _L
  mesh = plsc.VectorSubcoreMesh(core_axis_name="c", subcore_axis_name="s")
  out_t = tuple(jax.ShapeDtypeStruct((n,), jnp.float32) for _ in range(6))
  scratch = ([pltpu.VMEM((bpw * 4,), jnp.float32)]
             + [pltpu.VMEM((bpw,), jnp.float32) for _ in range(6)])

  @functools.partial(pl.kernel, mesh=mesh, out_type=out_t,
                     scratch_types=scratch)
  def k(bbs_hbm, ox_h, oy_h, ow_h, oh_h, otw_h, oth_h,
        buf, ox, oy, ow, oh, otw, oth):
    wid = lax.axis_index("s") * _NC + lax.axis_index("c")
    base = jnp.minimum(wid * bpw, n - bpw)
    base = pl.multiple_of(base, 8)
    pltpu.sync_copy(bbs_hbm.at[pl.ds(base * 4, bpw * 4)], buf)

    lane = jnp.arange(_L, dtype=jnp.int32)
    fidx = (lane % 4) * 4
    m0 = lane < 4
    m1 = lane < 8
    m2 = lane < 12

    def deint(v0, v1, v2, v3, f):
      idx = fidx + f
      g0 = _vgather(v0, idx)
      g1 = _vgather(v1, idx)
      g2 = _vgather(v2, idx)
      g3 = _vgather(v3, idx)
      return jnp.where(m0, g0, jnp.where(m1, g1, jnp.where(m2, g2, g3)))

    def body(g, carry):
      off = g * (4 * _L)
      v0 = buf[pl.ds(off, _L)]
      v1 = buf[pl.ds(off + _L, _L)]
      v2 = buf[pl.ds(off + 2 * _L, _L)]
      v3 = buf[pl.ds(off + 3 * _L, _L)]
      x = deint(v0, v1, v2, v3, 0)
      y = deint(v0, v1, v2, v3, 1)
      bw = deint(v0, v1, v2, v3, 2)
      bh = deint(v0, v1, v2, v3, 3)
      ix, iy, iw, ih, tw, th = _align16(x, y, bw, bh, H, W)
      s = pl.ds(g * _L, _L)
      ox[s] = ix
      oy[s] = iy
      ow[s] = iw
      oh[s] = ih
      otw[s] = tw
      oth[s] = th
      return carry

    lax.fori_loop(0, ngroups, body, 0)

    dst = pl.ds(base, bpw)
    pltpu.sync_copy(ox, ox_h.at[dst])
    pltpu.sync_copy(oy, oy_h.at[dst])
    pltpu.sync_copy(ow, ow_h.at[dst])
    pltpu.sync_copy(oh, oh_h.at[dst])
    pltpu.sync_copy(otw, otw_h.at[dst])
    pltpu.sync_copy(oth, oth_h.at[dst])

  return k


def kernel(img, bbs):
  H = float(img.shape[2])
  W = float(img.shape[3])
  n = bbs.shape[0]
  chunk = _NW * _L
  bpw = (-(-n // chunk)) * _L
  assert n % 8 == 0 and n >= bpw
  k = _make_sc_kernel(n, bpw, H, W)
  return k(bbs.reshape(-1))


# speedup vs baseline: 1.0012x; 1.0012x over previous
import functools

import jax
import jax.numpy as jnp
from jax import lax
from jax.experimental import pallas as pl
from jax.experimental.pallas import tpu as pltpu
from jax.experimental.pallas import tpu_sc as plsc

_L = 16
_NC = 1
_NS = 16
_NW = _NC * _NS

_EF = 1.5
_TS = 256.0
_ML = 3.0

_GDN = lax.GatherDimensionNumbers(
    offset_dims=(), collapsed_slice_dims=(0,), start_index_map=(0,))


def _vgather(v, idx):
  return lax.gather(v, idx[:, None], _GDN, (1,),
                    mode=lax.GatherScatterMode.PROMISE_IN_BOUNDS)


def _floorv(v):
  t = v.astype(jnp.int32).astype(jnp.float32)
  return jnp.where(t > v, t - 1.0, t)


def _ceilv(v):
  t = v.astype(jnp.int32).astype(jnp.float32)
  return jnp.where(t < v, t + 1.0, t)


def _align16(x, y, bw, bh, H, W):
  w = _ceilv(bw * _EF)
  h = _ceilv(bh * _EF)
  ix = _floorv(x - w * 0.5)
  cx = ix < 0.0
  w = jnp.where(cx, w + ix, w)
  ix = jnp.where(cx, 0.0, ix)
  iy = _floorv(y - h * 0.5)
  cy = iy < 0.0
  h = jnp.where(cy, h + iy, h)
  iy = jnp.where(cy, 0.0, iy)
  w = jnp.maximum(w, _ML)
  h = jnp.maximum(h, _ML)
  iw = W - ix
  iw = jnp.where(w < iw, w, iw)
  ih = H - iy
  ih = jnp.where(h < ih, h, ih)
  sx = iw < _ML
  iw = jnp.where(sx, _ML, iw)
  ix = jnp.where(sx, W - _ML, ix)
  sy = ih < _ML
  ih = jnp.where(sy, _ML, ih)
  iy = jnp.where(sy, H - _ML, iy)
  nidx = iw > ih
  th = jnp.where(nidx, _TS, _TS * ih / iw)
  tw = jnp.where(nidx, _TS * iw / ih, _TS)
  return ix, iy, iw, ih, tw, th


@functools.lru_cache(maxsize=None)
def _make_sc_kernel(n, bpw, H, W):
  ngroups = bpw // _L
  mesh = plsc.VectorSubcoreMesh(core_axis_name="c", subcore_axis_name="s",
                                num_cores=1)
  out_t = tuple(jax.ShapeDtypeStruct((n,), jnp.float32) for _ in range(6))
  scratch = ([pltpu.VMEM((bpw * 4,), jnp.float32)]
             + [pltpu.VMEM((bpw,), jnp.float32) for _ in range(6)])

  @functools.partial(pl.kernel, mesh=mesh, out_type=out_t,
                     scratch_types=scratch)
  def k(bbs_hbm, ox_h, oy_h, ow_h, oh_h, otw_h, oth_h,
        buf, ox, oy, ow, oh, otw, oth):
    wid = lax.axis_index("s") * _NC + lax.axis_index("c")
    base = jnp.minimum(wid * bpw, n - bpw)
    base = pl.multiple_of(base, 8)
    pltpu.sync_copy(bbs_hbm.at[pl.ds(base * 4, bpw * 4)], buf)

    lane = jnp.arange(_L, dtype=jnp.int32)
    fidx = (lane % 4) * 4
    m0 = lane < 4
    m1 = lane < 8
    m2 = lane < 12

    def deint(v0, v1, v2, v3, f):
      idx = fidx + f
      g0 = _vgather(v0, idx)
      g1 = _vgather(v1, idx)
      g2 = _vgather(v2, idx)
      g3 = _vgather(v3, idx)
      return jnp.where(m0, g0, jnp.where(m1, g1, jnp.where(m2, g2, g3)))

    def body(g, carry):
      off = g * (4 * _L)
      v0 = buf[pl.ds(off, _L)]
      v1 = buf[pl.ds(off + _L, _L)]
      v2 = buf[pl.ds(off + 2 * _L, _L)]
      v3 = buf[pl.ds(off + 3 * _L, _L)]
      x = deint(v0, v1, v2, v3, 0)
      y = deint(v0, v1, v2, v3, 1)
      bw = deint(v0, v1, v2, v3, 2)
      bh = deint(v0, v1, v2, v3, 3)
      ix, iy, iw, ih, tw, th = _align16(x, y, bw, bh, H, W)
      s = pl.ds(g * _L, _L)
      ox[s] = ix
      oy[s] = iy
      ow[s] = iw
      oh[s] = ih
      otw[s] = tw
      oth[s] = th
      return carry

    lax.fori_loop(0, ngroups, body, 0)

    dst = pl.ds(base, bpw)
    pltpu.sync_copy(ox, ox_h.at[dst])
    pltpu.sync_copy(oy, oy_h.at[dst])
    pltpu.sync_copy(ow, ow_h.at[dst])
    pltpu.sync_copy(oh, oh_h.at[dst])
    pltpu.sync_copy(otw, otw_h.at[dst])
    pltpu.sync_copy(oth, oth_h.at[dst])

  return k


def kernel(img, bbs):
  H = float(img.shape[2])
  W = float(img.shape[3])
  n = bbs.shape[0]
  chunk = _NW * _L
  bpw = (-(-n // chunk)) * _L
  assert n % 8 == 0 and n >= bpw
  k = _make_sc_kernel(n, bpw, H, W)
  return k(bbs.reshape(-1))


# speedup vs baseline: 1.5361x; 1.5342x over previous
import functools

import jax
import jax.numpy as jnp
from jax import lax
from jax.experimental import pallas as pl
from jax.experimental.pallas import tpu as pltpu
from jax.experimental.pallas import tpu_sc as plsc

_L = 16
_NC = 2
_NS = 16
_NW = _NC * _NS

_EF = 1.5
_TS = 256.0
_ML = 3.0


def _floorv(v):
  t = v.astype(jnp.int32).astype(jnp.float32)
  return jnp.where(t > v, t - 1.0, t)


def _ceilv(v):
  t = v.astype(jnp.int32).astype(jnp.float32)
  return jnp.where(t < v, t + 1.0, t)


def _align16(x, y, bw, bh, H, W):
  w = _ceilv(bw * _EF)
  h = _ceilv(bh * _EF)
  ix = _floorv(x - w * 0.5)
  cx = ix < 0.0
  w = jnp.where(cx, w + ix, w)
  ix = jnp.where(cx, 0.0, ix)
  iy = _floorv(y - h * 0.5)
  cy = iy < 0.0
  h = jnp.where(cy, h + iy, h)
  iy = jnp.where(cy, 0.0, iy)
  w = jnp.maximum(w, _ML)
  h = jnp.maximum(h, _ML)
  iw = W - ix
  iw = jnp.where(w < iw, w, iw)
  ih = H - iy
  ih = jnp.where(h < ih, h, ih)
  sx = iw < _ML
  iw = jnp.where(sx, _ML, iw)
  ix = jnp.where(sx, W - _ML, ix)
  sy = ih < _ML
  ih = jnp.where(sy, _ML, ih)
  iy = jnp.where(sy, H - _ML, iy)
  nidx = iw > ih
  th = jnp.where(nidx, _TS, _TS * ih / iw)
  tw = jnp.where(nidx, _TS * iw / ih, _TS)
  return ix, iy, iw, ih, tw, th


@functools.lru_cache(maxsize=None)
def _make_sc_kernel(n, bpw, H, W):
  ngroups = bpw // _L
  mesh = plsc.VectorSubcoreMesh(core_axis_name="c", subcore_axis_name="s",
                                num_cores=_NC)
  out_t = tuple(jax.ShapeDtypeStruct((n,), jnp.float32) for _ in range(6))
  scratch = [pltpu.VMEM((bpw,), jnp.float32) for _ in range(10)]

  @functools.partial(pl.kernel, mesh=mesh, out_type=out_t,
                     scratch_types=scratch)
  def k(x_h, y_h, w_h, h_h, ox_h, oy_h, ow_h, oh_h, otw_h, oth_h,
        xb, yb, wb, hb, ox, oy, ow, oh, otw, oth):
    wid = lax.axis_index("s") * _NC + lax.axis_index("c")
    base = jnp.minimum(wid * bpw, n - bpw)
    base = pl.multiple_of(base, 8)
    src = pl.ds(base, bpw)
    pltpu.sync_copy(x_h.at[src], xb)
    pltpu.sync_copy(y_h.at[src], yb)
    pltpu.sync_copy(w_h.at[src], wb)
    pltpu.sync_copy(h_h.at[src], hb)

    def body(g, carry):
      s = pl.ds(g * _L, _L)
      ix, iy, iw, ih, tw, th = _align16(xb[s], yb[s], wb[s], hb[s], H, W)
      ox[s] = ix
      oy[s] = iy
      ow[s] = iw
      oh[s] = ih
      otw[s] = tw
      oth[s] = th
      return carry

    lax.fori_loop(0, ngroups, body, 0)

    pltpu.sync_copy(ox, ox_h.at[src])
    pltpu.sync_copy(oy, oy_h.at[src])
    pltpu.sync_copy(ow, ow_h.at[src])
    pltpu.sync_copy(oh, oh_h.at[src])
    pltpu.sync_copy(otw, otw_h.at[src])
    pltpu.sync_copy(oth, oth_h.at[src])

  return k


def kernel(img, bbs):
  H = float(img.shape[2])
  W = float(img.shape[3])
  n = bbs.shape[0]
  chunk = _NW * _L
  bpw = (-(-n // chunk)) * _L
  assert n % 8 == 0 and n >= bpw
  k = _make_sc_kernel(n, bpw, H, W)
  return k(bbs[:, 0], bbs[:, 1], bbs[:, 2], bbs[:, 3])
